# baseline (speedup 1.0000x reference)
import functools

import jax
import jax.numpy as jnp
from jax import lax
from jax.experimental import pallas as pl
from jax.experimental.pallas import tpu as pltpu


def _stage_conv(xt, w_ref, pad_ref, col_ref, *, H, W, Cin):
    pad_ref[0:1, :, :] = jnp.zeros((1, W + 2, Cin), jnp.bfloat16)
    pad_ref[H + 1:H + 2, :, :] = jnp.zeros((1, W + 2, Cin), jnp.bfloat16)
    pad_ref[:, 0:1, :] = jnp.zeros((H + 2, 1, Cin), jnp.bfloat16)
    pad_ref[:, W + 1:W + 2, :] = jnp.zeros((H + 2, 1, Cin), jnp.bfloat16)
    pad_ref[1:H + 1, 1:W + 1, :] = xt.reshape(H, W, Cin)

    for t in range(9):
        dy, dx = divmod(t, 3)
        col_ref[:, :, t * Cin:(t + 1) * Cin] = pad_ref[dy:dy + H, dx:dx + W, :]

    return jnp.dot(col_ref[...].reshape(H * W, 9 * Cin), w_ref[...],
                   preferred_element_type=jnp.float32)


def _conv1_kernel(x_ref, w_ref, y_ref, stats_ref, pad_ref, col_ref,
                  *, H, W, Cin):
    xt = jnp.transpose(x_ref[0].astype(jnp.bfloat16))
    y = _stage_conv(xt, w_ref, pad_ref, col_ref, H=H, W=W, Cin=Cin)
    stats_ref[0, 0:1, :] = jnp.sum(y, axis=0, keepdims=True)
    stats_ref[0, 1:2, :] = jnp.sum(y * y, axis=0, keepdims=True)
    y_ref[0] = y.astype(jnp.bfloat16)


def _conv2_kernel(x_ref, w_ref, scale_ref, shift_ref, y_ref, stats_ref,
                  pad_ref, col_ref, *, H, W, Cin):
    xa = x_ref[0].astype(jnp.float32) * scale_ref[...] + shift_ref[...]
    xt = jnp.maximum(xa, 0.0).astype(jnp.bfloat16)
    y = _stage_conv(xt, w_ref, pad_ref, col_ref, H=H, W=W, Cin=Cin)
    stats_ref[0, 0:1, :] = jnp.sum(y, axis=0, keepdims=True)
    stats_ref[0, 1:2, :] = jnp.sum(y * y, axis=0, keepdims=True)
    y_ref[0] = jnp.transpose(y.astype(jnp.bfloat16))


def _final_kernel(y_ref, scale_ref, shift_ref, o_ref):
    y = y_ref[0].astype(jnp.float32)
    o_ref[0] = jnp.maximum(y * scale_ref[...] + shift_ref[...], 0.0)


def _conv1(x, w_slab, *, H, W, Cin, Cout):
    N = x.shape[0]
    HW = H * W
    kern = functools.partial(_conv1_kernel, H=H, W=W, Cin=Cin)
    return pl.pallas_call(
        kern,
        grid=(N,),
        in_specs=[
            pl.BlockSpec((1, Cin, HW), lambda n: (n, 0, 0)),
            pl.BlockSpec((9 * Cin, Cout), lambda n: (0, 0)),
        ],
        out_specs=(
            pl.BlockSpec((1, HW, Cout), lambda n: (n, 0, 0)),
            pl.BlockSpec((1, 8, Cout), lambda n: (n, 0, 0)),
        ),
        out_shape=(
            jax.ShapeDtypeStruct((N, HW, Cout), jnp.bfloat16),
            jax.ShapeDtypeStruct((N, 8, Cout), jnp.float32),
        ),
        scratch_shapes=[
            pltpu.VMEM((H + 2, W + 2, Cin), jnp.bfloat16),
            pltpu.VMEM((H, W, 9 * Cin), jnp.bfloat16),
        ],
        compiler_params=pltpu.CompilerParams(
            dimension_semantics=("parallel",)),
    )(x, w_slab)


def _conv2(x, w_slab, scale, shift, *, H, W, Cin, Cout):
    N = x.shape[0]
    HW = H * W
    kern = functools.partial(_conv2_kernel, H=H, W=W, Cin=Cin)
    return pl.pallas_call(
        kern,
        grid=(N,),
        in_specs=[
            pl.BlockSpec((1, HW, Cin), lambda n: (n, 0, 0)),
            pl.BlockSpec((9 * Cin, Cout), lambda n: (0, 0)),
            pl.BlockSpec((1, Cin), lambda n: (0, 0)),
            pl.BlockSpec((1, Cin), lambda n: (0, 0)),
        ],
        out_specs=(
            pl.BlockSpec((1, Cout, HW), lambda n: (n, 0, 0)),
            pl.BlockSpec((1, 8, Cout), lambda n: (n, 0, 0)),
        ),
        out_shape=(
            jax.ShapeDtypeStruct((N, Cout, HW), jnp.bfloat16),
            jax.ShapeDtypeStruct((N, 8, Cout), jnp.float32),
        ),
        scratch_shapes=[
            pltpu.VMEM((H + 2, W + 2, Cin), jnp.bfloat16),
            pltpu.VMEM((H, W, 9 * Cin), jnp.bfloat16),
        ],
        compiler_params=pltpu.CompilerParams(
            dimension_semantics=("parallel",)),
    )(x, w_slab, scale, shift)


def _final(y, scale, shift):
    N, Cout, HW = y.shape
    return pl.pallas_call(
        _final_kernel,
        grid=(N,),
        in_specs=[
            pl.BlockSpec((1, Cout, HW), lambda n: (n, 0, 0)),
            pl.BlockSpec((Cout, 1), lambda n: (0, 0)),
            pl.BlockSpec((Cout, 1), lambda n: (0, 0)),
        ],
        out_specs=pl.BlockSpec((1, Cout, HW), lambda n: (n, 0, 0)),
        out_shape=jax.ShapeDtypeStruct((N, Cout, HW), jnp.float32),
        compiler_params=pltpu.CompilerParams(
            dimension_semantics=("parallel",)),
    )(y, scale, shift)


def _bn_affine(stats, gamma, beta, count, eps=1e-5):
    s = jnp.sum(stats[:, 0, :], axis=0)
    ss = jnp.sum(stats[:, 1, :], axis=0)
    mean = s / count
    var = jnp.maximum(ss / count - mean * mean, 0.0)
    scale = gamma * lax.rsqrt(var + eps)
    shift = beta - mean * scale
    return scale, shift


def kernel(x_nchw, w1, cb1, g1, b1, w2, cb2, g2, b2):
    del cb1, cb2
    N, Cin, H, W = x_nchw.shape
    Cout = w1.shape[0]
    HW = H * W
    count = float(N * HW)

    def slab(w):
        wt = jnp.transpose(w, (2, 3, 1, 0))
        return wt.reshape(9 * w.shape[1], w.shape[0]).astype(jnp.bfloat16)

    x = x_nchw.reshape(N, Cin, HW)
    y1, st1 = _conv1(x, slab(w1), H=H, W=W, Cin=Cin, Cout=Cout)
    scale1, shift1 = _bn_affine(st1, g1, b1, count)
    y2, st2 = _conv2(y1, slab(w2), scale1.reshape(1, Cout),
                     shift1.reshape(1, Cout), H=H, W=W, Cin=Cout, Cout=Cout)
    scale2, shift2 = _bn_affine(st2, g2, b2, count)
    out = _final(y2, scale2.reshape(Cout, 1), shift2.reshape(Cout, 1))
    return out.reshape(N, Cout, H, W)


# speedup vs baseline: 1.3640x; 1.3640x over previous
import functools

import jax
import jax.numpy as jnp
from jax import lax
from jax.experimental import pallas as pl
from jax.experimental.pallas import tpu as pltpu


def _stage_conv(xt, w_ref, pad_ref, col_ref, *, H, W, Cin):
    pad_ref[0:1, :, :] = jnp.zeros((1, W + 2, Cin), jnp.bfloat16)
    pad_ref[H + 1:H + 2, :, :] = jnp.zeros((1, W + 2, Cin), jnp.bfloat16)
    pad_ref[:, 0:1, :] = jnp.zeros((H + 2, 1, Cin), jnp.bfloat16)
    pad_ref[:, W + 1:W + 2, :] = jnp.zeros((H + 2, 1, Cin), jnp.bfloat16)
    pad_ref[1:H + 1, 1:W + 1, :] = xt.reshape(H, W, Cin)

    for t in range(9):
        dy, dx = divmod(t, 3)
        col_ref[:, :, t * Cin:(t + 1) * Cin] = pad_ref[dy:dy + H, dx:dx + W, :]

    return jnp.dot(col_ref[...].reshape(H * W, 9 * Cin), w_ref[...],
                   preferred_element_type=jnp.float32)


def _conv1_kernel(x_ref, w_ref, y_ref, stats_ref, pad_ref, col_ref,
                  *, H, W, Cin):
    xt = x_ref[0].astype(jnp.bfloat16)
    y = _stage_conv(xt, w_ref, pad_ref, col_ref, H=H, W=W, Cin=Cin)
    stats_ref[0, 0:1, :] = jnp.sum(y, axis=0, keepdims=True)
    stats_ref[0, 1:2, :] = jnp.sum(y * y, axis=0, keepdims=True)
    y_ref[0] = y.astype(jnp.bfloat16)


def _conv2_kernel(x_ref, w_ref, scale_ref, shift_ref, y_ref, stats_ref,
                  pad_ref, col_ref, *, H, W, Cin):
    xa = x_ref[0].astype(jnp.float32) * scale_ref[...] + shift_ref[...]
    xt = jnp.maximum(xa, 0.0).astype(jnp.bfloat16)
    y = _stage_conv(xt, w_ref, pad_ref, col_ref, H=H, W=W, Cin=Cin)
    stats_ref[0, 0:1, :] = jnp.sum(y, axis=0, keepdims=True)
    stats_ref[0, 1:2, :] = jnp.sum(y * y, axis=0, keepdims=True)
    y_ref[0] = y.astype(jnp.bfloat16)


def _final_kernel(y_ref, scale_ref, shift_ref, o_ref):
    y = y_ref[0].astype(jnp.float32)
    o_ref[0] = jnp.maximum(y * scale_ref[...] + shift_ref[...], 0.0)


def _conv1(x, w_slab, *, H, W, Cin, Cout):
    N = x.shape[0]
    HW = H * W
    kern = functools.partial(_conv1_kernel, H=H, W=W, Cin=Cin)
    return pl.pallas_call(
        kern,
        grid=(N,),
        in_specs=[
            pl.BlockSpec((1, HW, Cin), lambda n: (n, 0, 0)),
            pl.BlockSpec((9 * Cin, Cout), lambda n: (0, 0)),
        ],
        out_specs=(
            pl.BlockSpec((1, HW, Cout), lambda n: (n, 0, 0)),
            pl.BlockSpec((1, 8, Cout), lambda n: (n, 0, 0)),
        ),
        out_shape=(
            jax.ShapeDtypeStruct((N, HW, Cout), jnp.bfloat16),
            jax.ShapeDtypeStruct((N, 8, Cout), jnp.float32),
        ),
        scratch_shapes=[
            pltpu.VMEM((H + 2, W + 2, Cin), jnp.bfloat16),
            pltpu.VMEM((H, W, 9 * Cin), jnp.bfloat16),
        ],
        compiler_params=pltpu.CompilerParams(
            dimension_semantics=("parallel",)),
    )(x, w_slab)


def _conv2(x, w_slab, scale, shift, *, H, W, Cin, Cout):
    N = x.shape[0]
    HW = H * W
    kern = functools.partial(_conv2_kernel, H=H, W=W, Cin=Cin)
    return pl.pallas_call(
        kern,
        grid=(N,),
        in_specs=[
            pl.BlockSpec((1, HW, Cin), lambda n: (n, 0, 0)),
            pl.BlockSpec((9 * Cin, Cout), lambda n: (0, 0)),
            pl.BlockSpec((1, Cin), lambda n: (0, 0)),
            pl.BlockSpec((1, Cin), lambda n: (0, 0)),
        ],
        out_specs=(
            pl.BlockSpec((1, HW, Cout), lambda n: (n, 0, 0)),
            pl.BlockSpec((1, 8, Cout), lambda n: (n, 0, 0)),
        ),
        out_shape=(
            jax.ShapeDtypeStruct((N, HW, Cout), jnp.bfloat16),
            jax.ShapeDtypeStruct((N, 8, Cout), jnp.float32),
        ),
        scratch_shapes=[
            pltpu.VMEM((H + 2, W + 2, Cin), jnp.bfloat16),
            pltpu.VMEM((H, W, 9 * Cin), jnp.bfloat16),
        ],
        compiler_params=pltpu.CompilerParams(
            dimension_semantics=("parallel",)),
    )(x, w_slab, scale, shift)


def _final(y, scale, shift):
    N, HW, Cout = y.shape
    return pl.pallas_call(
        _final_kernel,
        grid=(N,),
        in_specs=[
            pl.BlockSpec((1, HW, Cout), lambda n: (n, 0, 0)),
            pl.BlockSpec((1, Cout), lambda n: (0, 0)),
            pl.BlockSpec((1, Cout), lambda n: (0, 0)),
        ],
        out_specs=pl.BlockSpec((1, HW, Cout), lambda n: (n, 0, 0)),
        out_shape=jax.ShapeDtypeStruct((N, HW, Cout), jnp.float32),
        compiler_params=pltpu.CompilerParams(
            dimension_semantics=("parallel",)),
    )(y, scale, shift)


def _bn_affine(stats, gamma, beta, count, eps=1e-5):
    s = jnp.sum(stats[:, 0, :], axis=0)
    ss = jnp.sum(stats[:, 1, :], axis=0)
    mean = s / count
    var = jnp.maximum(ss / count - mean * mean, 0.0)
    scale = gamma * lax.rsqrt(var + eps)
    shift = beta - mean * scale
    cp = scale.shape[0]
    return scale.reshape(1, cp), shift.reshape(1, cp)


def kernel(x_nchw, w1, cb1, g1, b1, w2, cb2, g2, b2):
    del cb1, cb2
    N, Cin, H, W = x_nchw.shape
    Cout = w1.shape[0]
    HW = H * W
    count = float(N * HW)

    def slab(w):
        wt = jnp.transpose(w, (2, 3, 1, 0))
        return wt.reshape(9 * w.shape[1], w.shape[0]).astype(jnp.bfloat16)

    x_rows = jnp.transpose(x_nchw, (0, 2, 3, 1)).reshape(N, HW, Cin)

    y1, st1 = _conv1(x_rows, slab(w1), H=H, W=W, Cin=Cin, Cout=Cout)
    scale1, shift1 = _bn_affine(st1, g1, b1, count)
    y2, st2 = _conv2(y1, slab(w2), scale1, shift1,
                     H=H, W=W, Cin=Cout, Cout=Cout)
    scale2, shift2 = _bn_affine(st2, g2, b2, count)
    out = _final(y2, scale2, shift2)

    out = out.reshape(N, H, W, Cout)
    return jnp.transpose(out, (0, 3, 1, 2))
